# Initial kernel scaffold; baseline (speedup 1.0000x reference)
import jax
import jax.numpy as jnp
from jax.experimental import pallas as pl


def kernel(cdata, vdata, edata_e, vidx, eidx, W, b):
    raise NotImplementedError("write your pallas kernel here")


# speedup vs baseline: 5.9496x; 5.9496x over previous
import functools

import jax
import jax.numpy as jnp
from jax import lax
from jax.experimental import pallas as pl
from jax.experimental.pallas import tpu as pltpu
from jax.experimental.pallas import tpu_sc as plsc

NUM_GRAPHS = 100
N_VERT = 50000
N_EDGE = 1600000
D_V = 128
D_E = 16

NC, NS = 2, 16
NW = NC * NS

CV = 400
NV_CHUNKS = N_VERT // CV
CV_SUB = 80
NV_SUB = CV // CV_SUB
NV_ITERS = -(-NV_CHUNKS // NW)

CE = 2560
NE_CHUNKS = N_EDGE // CE
CE_SUB = 128
NE_SUB = CE // CE_SUB
NE_ITERS = -(-NE_CHUNKS // NW)


def _sc_segment_sums(vdata, vidx3, edata, eidx3):
    mesh = plsc.VectorSubcoreMesh(core_axis_name="c", subcore_axis_name="s")

    @functools.partial(
        pl.kernel,
        mesh=mesh,
        compiler_params=pltpu.CompilerParams(use_tc_tiling_on_sc=False),
        out_type=(
            jax.ShapeDtypeStruct((NC, NUM_GRAPHS, D_V), jnp.float32),
            jax.ShapeDtypeStruct((NC, NUM_GRAPHS, D_E), jnp.float32),
            jax.ShapeDtypeStruct((NC, NUM_GRAPHS, D_E), jnp.float32),
        ),
        scratch_types=(
            pltpu.VMEM((CV, D_V), jnp.float32),
            pltpu.VMEM((NV_SUB, CV_SUB), jnp.int32),
            pltpu.VMEM((CE, D_E), jnp.float32),
            pltpu.VMEM((NE_SUB, CE_SUB), jnp.int32),
            pltpu.VMEM((CV_SUB, D_E), jnp.float32),
            pltpu.VMEM_SHARED((NUM_GRAPHS, D_V), jnp.float32),
            pltpu.VMEM_SHARED((NUM_GRAPHS, D_E), jnp.float32),
            pltpu.VMEM_SHARED((NUM_GRAPHS, D_E), jnp.float32),
        ),
    )
    def k(vdata_h, vidx_h, edata_h, eidx_h, vout, cout, eout,
          vbuf, vibuf, ebuf, eibuf, ones, v_acc, c_acc, e_acc):
        c = lax.axis_index("c")
        s = lax.axis_index("s")
        wid = s * NC + c

        one = jnp.full((16,), 1.0, jnp.float32)
        zero = jnp.zeros((16,), jnp.float32)

        def ones_row(r, carry):
            ones[r, pl.ds(0, 16)] = one
            return carry

        lax.fori_loop(0, CV_SUB, ones_row, 0)

        @pl.when(s == 0)
        def _():
            def zv(r, carry):
                for j in range(D_V // 16):
                    vbuf[r, pl.ds(j * 16, 16)] = zero
                return carry

            lax.fori_loop(0, NUM_GRAPHS, zv, 0)

            def ze(r, carry):
                ebuf[r, pl.ds(0, 16)] = zero
                return carry

            lax.fori_loop(0, NUM_GRAPHS, ze, 0)
            pltpu.sync_copy(vbuf.at[pl.ds(0, NUM_GRAPHS)], v_acc)
            pltpu.sync_copy(ebuf.at[pl.ds(0, NUM_GRAPHS)], c_acc)
            pltpu.sync_copy(ebuf.at[pl.ds(0, NUM_GRAPHS)], e_acc)

        plsc.subcore_barrier()

        def vchunk(kk, carry):
            chunk = wid + NW * kk

            @pl.when(chunk < NV_CHUNKS)
            def _():
                pltpu.sync_copy(vdata_h.at[pl.ds(chunk * CV, CV)], vbuf)
                pltpu.sync_copy(vidx_h.at[chunk], vibuf)
                for j in range(NV_SUB):
                    idx = vibuf.at[j]
                    pltpu.sync_copy(
                        vbuf.at[pl.ds(j * CV_SUB, CV_SUB)],
                        v_acc.at[idx], add=True)
                    pltpu.sync_copy(ones, c_acc.at[idx], add=True)

            return carry

        lax.fori_loop(0, NV_ITERS, vchunk, 0)

        def echunk(kk, carry):
            chunk = wid + NW * kk

            @pl.when(chunk < NE_CHUNKS)
            def _():
                pltpu.sync_copy(edata_h.at[pl.ds(chunk * CE, CE)], ebuf)
                pltpu.sync_copy(eidx_h.at[chunk], eibuf)
                for j in range(NE_SUB):
                    idx = eibuf.at[j]
                    pltpu.sync_copy(
                        ebuf.at[pl.ds(j * CE_SUB, CE_SUB)],
                        e_acc.at[idx], add=True)

            return carry

        lax.fori_loop(0, NE_ITERS, echunk, 0)

        plsc.subcore_barrier()

        @pl.when(s == 0)
        def _():
            pltpu.sync_copy(v_acc, vout.at[c])
            pltpu.sync_copy(c_acc, cout.at[c])
            pltpu.sync_copy(e_acc, eout.at[c])

    return k(vdata, vidx3, edata, eidx3)


def _tc_final(cdata, vpart, cpart, epart, W, b2):
    def body(c_ref, v_ref, cnt_ref, e_ref, w_ref, b_ref, o_ref):
        v_sum = v_ref[0] + v_ref[1]
        cnt = cnt_ref[0] + cnt_ref[1]
        e_agg = e_ref[0] + e_ref[1]
        denom = jnp.maximum(jnp.max(cnt, axis=1, keepdims=True), 1.0)
        v_agg = v_sum / denom
        out = (
            jnp.dot(c_ref[...], w_ref[0:D_V, :],
                    preferred_element_type=jnp.float32)
            + jnp.dot(v_agg, w_ref[D_V:2 * D_V, :],
                      preferred_element_type=jnp.float32)
            + jnp.dot(e_agg, w_ref[2 * D_V:2 * D_V + D_E, :],
                      preferred_element_type=jnp.float32)
            + b_ref[...]
        )
        o_ref[...] = out

    return pl.pallas_call(
        body,
        out_shape=jax.ShapeDtypeStruct((NUM_GRAPHS, 128), jnp.float32),
    )(cdata, vpart, cpart, epart, W, b2)


def kernel(cdata, vdata, edata_e, vidx, eidx, W, b):
    vidx3 = vidx.reshape(NV_CHUNKS, NV_SUB, CV_SUB)
    eidx3 = eidx.reshape(NE_CHUNKS, NE_SUB, CE_SUB)
    vp, cp, ep = _sc_segment_sums(vdata, vidx3, edata_e, eidx3)
    return _tc_final(cdata, vp, cp, ep, W, b.reshape(1, -1))


# speedup vs baseline: 5.9663x; 1.0028x over previous
import functools

import jax
import jax.numpy as jnp
from jax import lax
from jax.experimental import pallas as pl
from jax.experimental.pallas import tpu as pltpu
from jax.experimental.pallas import tpu_sc as plsc

NUM_GRAPHS = 100
N_VERT = 50000
N_EDGE = 1600000
D_V = 128
D_E = 16

NC, NS = 2, 16
NW = NC * NS

CV = 400
NV_CHUNKS = N_VERT // CV
CV_SUB = 80
NV_SUB = CV // CV_SUB
NV_ITERS = -(-NV_CHUNKS // NW)

CE = 2560
NE_CHUNKS = N_EDGE // CE
CE_SUB = 128
NE_SUB = CE // CE_SUB
NE_ITERS = -(-NE_CHUNKS // NW)


def _sc_segment_sums(vdata, vidx, edata, eidx):
    mesh = plsc.VectorSubcoreMesh(core_axis_name="c", subcore_axis_name="s")

    @functools.partial(
        pl.kernel,
        mesh=mesh,
        compiler_params=pltpu.CompilerParams(use_tc_tiling_on_sc=False),
        out_type=(
            jax.ShapeDtypeStruct((NC, NUM_GRAPHS, D_V), jnp.float32),
            jax.ShapeDtypeStruct((NC, NUM_GRAPHS, D_E), jnp.float32),
            jax.ShapeDtypeStruct((NC, NUM_GRAPHS, D_E), jnp.float32),
        ),
        scratch_types=(
            pltpu.VMEM((CV, D_V), jnp.float32),
            pltpu.VMEM((CV,), jnp.int32),
            pltpu.VMEM((NV_SUB, CV_SUB), jnp.int32),
            pltpu.VMEM((CE, D_E), jnp.float32),
            pltpu.VMEM((CE,), jnp.int32),
            pltpu.VMEM((NE_SUB, CE_SUB), jnp.int32),
            pltpu.VMEM((CV_SUB, D_E), jnp.float32),
            pltpu.VMEM_SHARED((NUM_GRAPHS, D_V), jnp.float32),
            pltpu.VMEM_SHARED((NUM_GRAPHS, D_E), jnp.float32),
            pltpu.VMEM_SHARED((NUM_GRAPHS, D_E), jnp.float32),
        ),
    )
    def k(vdata_h, vidx_h, edata_h, eidx_h, vout, cout, eout,
          vbuf, vflat, vibuf, ebuf, eflat, eibuf, ones, v_acc, c_acc, e_acc):
        c = lax.axis_index("c")
        s = lax.axis_index("s")
        wid = s * NC + c

        one = jnp.full((16,), 1.0, jnp.float32)
        zero = jnp.zeros((16,), jnp.float32)

        def ones_row(r, carry):
            ones[r, pl.ds(0, 16)] = one
            return carry

        lax.fori_loop(0, CV_SUB, ones_row, 0)

        @pl.when(s == 0)
        def _():
            def zv(r, carry):
                for j in range(D_V // 16):
                    vbuf[r, pl.ds(j * 16, 16)] = zero
                return carry

            lax.fori_loop(0, NUM_GRAPHS, zv, 0)

            def ze(r, carry):
                ebuf[r, pl.ds(0, 16)] = zero
                return carry

            lax.fori_loop(0, NUM_GRAPHS, ze, 0)
            pltpu.sync_copy(vbuf.at[pl.ds(0, NUM_GRAPHS)], v_acc)
            pltpu.sync_copy(ebuf.at[pl.ds(0, NUM_GRAPHS)], c_acc)
            pltpu.sync_copy(ebuf.at[pl.ds(0, NUM_GRAPHS)], e_acc)

        plsc.subcore_barrier()

        def vchunk(kk, carry):
            chunk = wid + NW * kk

            @pl.when(chunk < NV_CHUNKS)
            def _():
                pltpu.sync_copy(vdata_h.at[pl.ds(chunk * CV, CV)], vbuf)
                pltpu.sync_copy(vidx_h.at[pl.ds(chunk * CV, CV)], vflat)
                for j in range(NV_SUB):
                    for l in range(CV_SUB // 16):
                        vibuf[j, pl.ds(l * 16, 16)] = vflat[
                            pl.ds(j * CV_SUB + l * 16, 16)]
                for j in range(NV_SUB):
                    idx = vibuf.at[j]
                    pltpu.sync_copy(
                        vbuf.at[pl.ds(j * CV_SUB, CV_SUB)],
                        v_acc.at[idx], add=True)
                    pltpu.sync_copy(ones, c_acc.at[idx], add=True)

            return carry

        lax.fori_loop(0, NV_ITERS, vchunk, 0)

        def echunk(kk, carry):
            chunk = wid + NW * kk

            @pl.when(chunk < NE_CHUNKS)
            def _():
                pltpu.sync_copy(edata_h.at[pl.ds(chunk * CE, CE)], ebuf)
                pltpu.sync_copy(eidx_h.at[pl.ds(chunk * CE, CE)], eflat)
                for j in range(NE_SUB):
                    for l in range(CE_SUB // 16):
                        eibuf[j, pl.ds(l * 16, 16)] = eflat[
                            pl.ds(j * CE_SUB + l * 16, 16)]
                for j in range(NE_SUB):
                    idx = eibuf.at[j]
                    pltpu.sync_copy(
                        ebuf.at[pl.ds(j * CE_SUB, CE_SUB)],
                        e_acc.at[idx], add=True)

            return carry

        lax.fori_loop(0, NE_ITERS, echunk, 0)

        plsc.subcore_barrier()

        @pl.when(s == 0)
        def _():
            pltpu.sync_copy(v_acc, vout.at[c])
            pltpu.sync_copy(c_acc, cout.at[c])
            pltpu.sync_copy(e_acc, eout.at[c])

    return k(vdata, vidx, edata, eidx)


def _tc_final(cdata, vpart, cpart, epart, W, b2):
    def body(c_ref, v_ref, cnt_ref, e_ref, w_ref, b_ref, o_ref):
        v_sum = v_ref[0] + v_ref[1]
        cnt = cnt_ref[0] + cnt_ref[1]
        e_agg = e_ref[0] + e_ref[1]
        denom = jnp.maximum(jnp.max(cnt, axis=1, keepdims=True), 1.0)
        v_agg = v_sum / denom
        out = (
            jnp.dot(c_ref[...], w_ref[0:D_V, :],
                    preferred_element_type=jnp.float32)
            + jnp.dot(v_agg, w_ref[D_V:2 * D_V, :],
                      preferred_element_type=jnp.float32)
            + jnp.dot(e_agg, w_ref[2 * D_V:2 * D_V + D_E, :],
                      preferred_element_type=jnp.float32)
            + b_ref[...]
        )
        o_ref[...] = out

    return pl.pallas_call(
        body,
        out_shape=jax.ShapeDtypeStruct((NUM_GRAPHS, 128), jnp.float32),
    )(cdata, vpart, cpart, epart, W, b2)


def kernel(cdata, vdata, edata_e, vidx, eidx, W, b):
    vp, cp, ep = _sc_segment_sums(vdata, vidx, edata_e, eidx)
    return _tc_final(cdata, vp, cp, ep, W, b.reshape(1, -1))


# speedup vs baseline: 8.7501x; 1.4666x over previous
import functools

import jax
import jax.numpy as jnp
from jax import lax
from jax.experimental import pallas as pl
from jax.experimental.pallas import tpu as pltpu
from jax.experimental.pallas import tpu_sc as plsc

NUM_GRAPHS = 100
N_VERT = 50000
N_EDGE = 1600000
D_V = 128
D_E = 16

NC, NS = 2, 16
NW = NC * NS

CV = 400
NV_CHUNKS = N_VERT // CV
CV_SUB = 80
NV_SUB = CV // CV_SUB
NV_ITERS = -(-NV_CHUNKS // NW)

KE = 2560
NE_CHUNKS = N_EDGE // KE


def _sc_vertex_sums(vdata, vidx):
    mesh = plsc.VectorSubcoreMesh(core_axis_name="c", subcore_axis_name="s")

    @functools.partial(
        pl.kernel,
        mesh=mesh,
        compiler_params=pltpu.CompilerParams(use_tc_tiling_on_sc=False),
        out_type=(
            jax.ShapeDtypeStruct((NC, NUM_GRAPHS, D_V), jnp.float32),
            jax.ShapeDtypeStruct((NC, NUM_GRAPHS, D_E), jnp.float32),
        ),
        scratch_types=(
            pltpu.VMEM((CV, D_V), jnp.float32),
            pltpu.VMEM((CV,), jnp.int32),
            pltpu.VMEM((NV_SUB, CV_SUB), jnp.int32),
            pltpu.VMEM((NUM_GRAPHS, D_E), jnp.float32),
            pltpu.VMEM((CV_SUB, D_E), jnp.float32),
            pltpu.VMEM_SHARED((NUM_GRAPHS, D_V), jnp.float32),
            pltpu.VMEM_SHARED((NUM_GRAPHS, D_E), jnp.float32),
        ),
    )
    def k(vdata_h, vidx_h, vout, cout,
          vbuf, vflat, vibuf, zbuf, ones, v_acc, c_acc):
        c = lax.axis_index("c")
        s = lax.axis_index("s")
        wid = s * NC + c

        one = jnp.full((16,), 1.0, jnp.float32)
        zero = jnp.zeros((16,), jnp.float32)

        def ones_row(r, carry):
            ones[r, pl.ds(0, 16)] = one
            return carry

        lax.fori_loop(0, CV_SUB, ones_row, 0)

        @pl.when(s == 0)
        def _():
            def zv(r, carry):
                for j in range(D_V // 16):
                    vbuf[r, pl.ds(j * 16, 16)] = zero
                return carry

            lax.fori_loop(0, NUM_GRAPHS, zv, 0)

            def ze(r, carry):
                zbuf[r, pl.ds(0, 16)] = zero
                return carry

            lax.fori_loop(0, NUM_GRAPHS, ze, 0)
            pltpu.sync_copy(vbuf.at[pl.ds(0, NUM_GRAPHS)], v_acc)
            pltpu.sync_copy(zbuf, c_acc)

        plsc.subcore_barrier()

        def vchunk(kk, carry):
            chunk = wid + NW * kk

            @pl.when(chunk < NV_CHUNKS)
            def _():
                pltpu.sync_copy(vdata_h.at[pl.ds(chunk * CV, CV)], vbuf)
                pltpu.sync_copy(vidx_h.at[pl.ds(chunk * CV, CV)], vflat)
                for j in range(NV_SUB):
                    for l in range(CV_SUB // 16):
                        vibuf[j, pl.ds(l * 16, 16)] = vflat[
                            pl.ds(j * CV_SUB + l * 16, 16)]
                for j in range(NV_SUB):
                    idx = vibuf.at[j]
                    pltpu.sync_copy(
                        vbuf.at[pl.ds(j * CV_SUB, CV_SUB)],
                        v_acc.at[idx], add=True)
                    pltpu.sync_copy(ones, c_acc.at[idx], add=True)

            return carry

        lax.fori_loop(0, NV_ITERS, vchunk, 0)

        plsc.subcore_barrier()

        @pl.when(s == 0)
        def _():
            pltpu.sync_copy(v_acc, vout.at[c])
            pltpu.sync_copy(c_acc, cout.at[c])

    return k(vdata, vidx)


def _tc_edge_sums(edata_t, seg_first, seg_last, starts2d, ends2d):
    def body(first_ref, last_ref, e_ref, st_ref, en_ref, acc_ref):
        ones_row = jnp.ones((1, KE), jnp.float32)
        ci = pl.program_id(0)

        @pl.when(ci == 0)
        def _():
            acc_ref[...] = jnp.zeros_like(acc_ref)

        s0 = first_ref[ci]
        s1 = last_ref[ci]
        blk = e_ref[...]

        @pl.when(s0 == s1)
        def _():
            row = lax.dot_general(
                ones_row, blk, (((1,), (1,)), ((), ())),
                precision=lax.Precision.HIGHEST,
                preferred_element_type=jnp.float32)
            sel = (lax.broadcasted_iota(jnp.int32, (NUM_GRAPHS, 1), 0)
                   == s0).astype(jnp.float32)
            acc_ref[...] += sel * row

        @pl.when(s0 != s1)
        def _():
            base = ci * KE
            pos = lax.broadcasted_iota(jnp.int32, (NUM_GRAPHS, KE), 1) + base
            m = jnp.logical_and(pos >= st_ref[...], pos < en_ref[...])
            mf = m.astype(jnp.float32)
            upd = lax.dot_general(
                mf, blk, (((1,), (1,)), ((), ())),
                precision=lax.Precision.HIGHEST,
                preferred_element_type=jnp.float32)
            acc_ref[...] += upd

    grid_spec = pltpu.PrefetchScalarGridSpec(
        num_scalar_prefetch=2,
        grid=(NE_CHUNKS,),
        in_specs=[
            pl.BlockSpec((D_E, KE), lambda ci, *_: (0, ci)),
            pl.BlockSpec((NUM_GRAPHS, 1), lambda ci, *_: (0, 0)),
            pl.BlockSpec((NUM_GRAPHS, 1), lambda ci, *_: (0, 0)),
        ],
        out_specs=pl.BlockSpec((NUM_GRAPHS, D_E), lambda ci, *_: (0, 0)),
    )
    return pl.pallas_call(
        body,
        grid_spec=grid_spec,
        out_shape=jax.ShapeDtypeStruct((NUM_GRAPHS, D_E), jnp.float32),
    )(seg_first, seg_last, edata_t, starts2d, ends2d)


def _tc_final(cdata, vpart, cpart, e_agg, W, b2):
    def body(c_ref, v_ref, cnt_ref, e_ref, w_ref, b_ref, o_ref):
        v_sum = v_ref[0] + v_ref[1]
        cnt = cnt_ref[0] + cnt_ref[1]
        denom = jnp.maximum(jnp.max(cnt, axis=1, keepdims=True), 1.0)
        v_agg = v_sum / denom
        out = (
            jnp.dot(c_ref[...], w_ref[0:D_V, :],
                    preferred_element_type=jnp.float32)
            + jnp.dot(v_agg, w_ref[D_V:2 * D_V, :],
                      preferred_element_type=jnp.float32)
            + jnp.dot(e_ref[...], w_ref[2 * D_V:2 * D_V + D_E, :],
                      preferred_element_type=jnp.float32)
            + b_ref[...]
        )
        o_ref[...] = out

    return pl.pallas_call(
        body,
        out_shape=jax.ShapeDtypeStruct((NUM_GRAPHS, 128), jnp.float32),
    )(cdata, vpart, cpart, e_agg, W, b2)


def kernel(cdata, vdata, edata_e, vidx, eidx, W, b):
    edata_t = edata_e.T
    seg_ids = jnp.arange(NUM_GRAPHS, dtype=eidx.dtype)
    starts = jnp.searchsorted(eidx, seg_ids).astype(jnp.int32)
    ends = jnp.concatenate(
        [starts[1:], jnp.full((1,), N_EDGE, jnp.int32)])
    starts2d = starts.reshape(NUM_GRAPHS, 1)
    ends2d = ends.reshape(NUM_GRAPHS, 1)
    seg_first = eidx[::KE].astype(jnp.int32)
    seg_last = eidx[KE - 1::KE].astype(jnp.int32)

    vp, cp = _sc_vertex_sums(vdata, vidx)
    e_agg = _tc_edge_sums(edata_t, seg_first, seg_last, starts2d, ends2d)
    return _tc_final(cdata, vp, cp, e_agg, W, b.reshape(1, -1))


# speedup vs baseline: 13.2649x; 1.5160x over previous
import functools

import jax
import jax.numpy as jnp
from jax import lax
from jax.experimental import pallas as pl
from jax.experimental.pallas import tpu as pltpu
from jax.experimental.pallas import tpu_sc as plsc

NUM_GRAPHS = 100
N_VERT = 50000
N_EDGE = 1600000
D_V = 128
D_E = 16

NC, NS = 2, 16
NW = NC * NS

CV = 400
NV_CHUNKS = N_VERT // CV
CV_SUB = 80
NV_SUB = CV // CV_SUB
NV_ITERS = -(-NV_CHUNKS // NW)

KE = 2560
NE_CHUNKS = N_EDGE // KE
U = 5
KB = KE * U
NB = NE_CHUNKS // U
KROWS = KE // 128
MAXM = 100


def _sc_vertex_sums(vdata, vidx):
    mesh = plsc.VectorSubcoreMesh(core_axis_name="c", subcore_axis_name="s")

    @functools.partial(
        pl.kernel,
        mesh=mesh,
        compiler_params=pltpu.CompilerParams(use_tc_tiling_on_sc=False),
        out_type=(
            jax.ShapeDtypeStruct((NC, NUM_GRAPHS, D_V), jnp.float32),
            jax.ShapeDtypeStruct((NC, NUM_GRAPHS, D_E), jnp.float32),
        ),
        scratch_types=(
            pltpu.VMEM((CV, D_V), jnp.float32),
            pltpu.VMEM((CV,), jnp.int32),
            pltpu.VMEM((NV_SUB, CV_SUB), jnp.int32),
            pltpu.VMEM((NUM_GRAPHS, D_E), jnp.float32),
            pltpu.VMEM((CV_SUB, D_E), jnp.float32),
            pltpu.VMEM_SHARED((NUM_GRAPHS, D_V), jnp.float32),
            pltpu.VMEM_SHARED((NUM_GRAPHS, D_E), jnp.float32),
        ),
    )
    def k(vdata_h, vidx_h, vout, cout,
          vbuf, vflat, vibuf, zbuf, ones, v_acc, c_acc):
        c = lax.axis_index("c")
        s = lax.axis_index("s")
        wid = s * NC + c

        one = jnp.full((16,), 1.0, jnp.float32)
        zero = jnp.zeros((16,), jnp.float32)

        def ones_row(r, carry):
            ones[r, pl.ds(0, 16)] = one
            return carry

        lax.fori_loop(0, CV_SUB, ones_row, 0)

        @pl.when(s == 0)
        def _():
            def zv(r, carry):
                for j in range(D_V // 16):
                    vbuf[r, pl.ds(j * 16, 16)] = zero
                return carry

            lax.fori_loop(0, NUM_GRAPHS, zv, 0)

            def ze(r, carry):
                zbuf[r, pl.ds(0, 16)] = zero
                return carry

            lax.fori_loop(0, NUM_GRAPHS, ze, 0)
            pltpu.sync_copy(vbuf.at[pl.ds(0, NUM_GRAPHS)], v_acc)
            pltpu.sync_copy(zbuf, c_acc)

        plsc.subcore_barrier()

        def vchunk(kk, carry):
            chunk = wid + NW * kk

            @pl.when(chunk < NV_CHUNKS)
            def _():
                pltpu.sync_copy(vdata_h.at[pl.ds(chunk * CV, CV)], vbuf)
                pltpu.sync_copy(vidx_h.at[pl.ds(chunk * CV, CV)], vflat)
                for j in range(NV_SUB):
                    for l in range(CV_SUB // 16):
                        vibuf[j, pl.ds(l * 16, 16)] = vflat[
                            pl.ds(j * CV_SUB + l * 16, 16)]
                for j in range(NV_SUB):
                    idx = vibuf.at[j]
                    pltpu.sync_copy(
                        vbuf.at[pl.ds(j * CV_SUB, CV_SUB)],
                        v_acc.at[idx], add=True)
                    pltpu.sync_copy(ones, c_acc.at[idx], add=True)

            return carry

        lax.fori_loop(0, NV_ITERS, vchunk, 0)

        plsc.subcore_barrier()

        @pl.when(s == 0)
        def _():
            pltpu.sync_copy(v_acc, vout.at[c])
            pltpu.sync_copy(c_acc, cout.at[c])

    return k(vdata, vidx)


def _tc_edge_full(edata_t, seg_first, seg_last):
    def body(first_ref, last_ref, e_ref, acc_ref):
        bi = pl.program_id(0)

        @pl.when(bi == 0)
        def _():
            acc_ref[...] = jnp.zeros_like(acc_ref)

        ones_row = jnp.ones((1, KE), jnp.float32)
        iota = lax.broadcasted_iota(jnp.int32, (NUM_GRAPHS, 1), 0)
        upd = jnp.zeros((NUM_GRAPHS, D_E), jnp.float32)
        for u in range(U):
            ci = bi * U + u
            s0 = first_ref[ci]
            s1 = last_ref[ci]
            blk = e_ref[:, u * KE:(u + 1) * KE]
            row = lax.dot_general(
                ones_row, blk, (((1,), (1,)), ((), ())),
                precision=lax.Precision.HIGHEST,
                preferred_element_type=jnp.float32)
            sel = jnp.where(s0 == s1, (iota == s0).astype(jnp.float32),
                            jnp.zeros((NUM_GRAPHS, 1), jnp.float32))
            upd = upd + sel * row
        acc_ref[...] += upd

    grid_spec = pltpu.PrefetchScalarGridSpec(
        num_scalar_prefetch=2,
        grid=(NB,),
        in_specs=[
            pl.BlockSpec((D_E, KB), lambda bi, *_: (0, bi)),
        ],
        out_specs=pl.BlockSpec((NUM_GRAPHS, D_E), lambda bi, *_: (0, 0)),
    )
    return pl.pallas_call(
        body,
        grid_spec=grid_spec,
        out_shape=jax.ShapeDtypeStruct((NUM_GRAPHS, D_E), jnp.float32),
    )(seg_first, seg_last, edata_t)


def _tc_edge_mixed(edata_t, eidx3, mixed_ids, n_mixed):
    def body(mid_ref, n_ref, e_ref, idx_ref, acc_ref):
        i = pl.program_id(0)

        @pl.when(i == 0)
        def _():
            acc_ref[...] = jnp.zeros_like(acc_ref)

        valid = i < n_ref[0]
        iota = lax.broadcasted_iota(jnp.int32, (NUM_GRAPHS, 1), 0)
        upd = jnp.zeros((NUM_GRAPHS, D_E), jnp.float32)
        for j in range(KROWS):
            idrow = idx_ref[:, j, :]
            m = jnp.where(valid, (iota == idrow).astype(jnp.float32),
                          jnp.zeros((NUM_GRAPHS, 128), jnp.float32))
            blkj = e_ref[:, j * 128:(j + 1) * 128]
            upd = upd + lax.dot_general(
                m, blkj, (((1,), (1,)), ((), ())),
                precision=lax.Precision.HIGHEST,
                preferred_element_type=jnp.float32)
        acc_ref[...] += upd

    grid_spec = pltpu.PrefetchScalarGridSpec(
        num_scalar_prefetch=2,
        grid=(MAXM,),
        in_specs=[
            pl.BlockSpec((D_E, KE), lambda i, mid, n: (0, mid[i])),
            pl.BlockSpec((1, KROWS, 128), lambda i, mid, n: (mid[i], 0, 0)),
        ],
        out_specs=pl.BlockSpec((NUM_GRAPHS, D_E), lambda i, *_: (0, 0)),
    )
    return pl.pallas_call(
        body,
        grid_spec=grid_spec,
        out_shape=jax.ShapeDtypeStruct((NUM_GRAPHS, D_E), jnp.float32),
    )(mixed_ids, n_mixed, edata_t, eidx3)


def _tc_final(cdata, vpart, cpart, e_full, e_mix, W, b2):
    def body(c_ref, v_ref, cnt_ref, ef_ref, em_ref, w_ref, b_ref, o_ref):
        v_sum = v_ref[0] + v_ref[1]
        cnt = cnt_ref[0] + cnt_ref[1]
        denom = jnp.maximum(jnp.max(cnt, axis=1, keepdims=True), 1.0)
        v_agg = v_sum / denom
        e_agg = ef_ref[...] + em_ref[...]
        out = (
            jnp.dot(c_ref[...], w_ref[0:D_V, :],
                    preferred_element_type=jnp.float32)
            + jnp.dot(v_agg, w_ref[D_V:2 * D_V, :],
                      preferred_element_type=jnp.float32)
            + jnp.dot(e_agg, w_ref[2 * D_V:2 * D_V + D_E, :],
                      preferred_element_type=jnp.float32)
            + b_ref[...]
        )
        o_ref[...] = out

    return pl.pallas_call(
        body,
        out_shape=jax.ShapeDtypeStruct((NUM_GRAPHS, 128), jnp.float32),
    )(cdata, vpart, cpart, e_full, e_mix, W, b2)


def kernel(cdata, vdata, edata_e, vidx, eidx, W, b):
    edata_t = edata_e.T
    seg_first = eidx[::KE].astype(jnp.int32)
    seg_last = eidx[KE - 1::KE].astype(jnp.int32)
    mixed = seg_first != seg_last
    mixed_ids = jnp.nonzero(mixed, size=MAXM, fill_value=0)[0].astype(
        jnp.int32)
    n_mixed = jnp.sum(mixed.astype(jnp.int32)).reshape(1)
    eidx3 = eidx.reshape(NE_CHUNKS, KROWS, 128)

    vp, cp = _sc_vertex_sums(vdata, vidx)
    e_full = _tc_edge_full(edata_t, seg_first, seg_last)
    e_mix = _tc_edge_mixed(edata_t, eidx3, mixed_ids, n_mixed)
    return _tc_final(cdata, vp, cp, e_full, e_mix, W, b.reshape(1, -1))


# speedup vs baseline: 21.0700x; 1.5884x over previous
import functools

import jax
import jax.numpy as jnp
from jax import lax
from jax.experimental import pallas as pl
from jax.experimental.pallas import tpu as pltpu
from jax.experimental.pallas import tpu_sc as plsc

NUM_GRAPHS = 100
N_VERT = 50000
N_EDGE = 1600000
D_V = 128
D_E = 16

NC, NS = 2, 16
NW = NC * NS

CV = 400
NV_CHUNKS = N_VERT // CV
CV_SUB = 80
NV_SUB = CV // CV_SUB
NV_ITERS = -(-NV_CHUNKS // NW)

KE = 2560
NE_CHUNKS = N_EDGE // KE
U = 5
KB = KE * U
NB = NE_CHUNKS // U
KROWS = KE // 128
MAXM = 100


def _sc_vertex_sums(vdata, vidx):
    mesh = plsc.VectorSubcoreMesh(core_axis_name="c", subcore_axis_name="s")

    @functools.partial(
        pl.kernel,
        mesh=mesh,
        compiler_params=pltpu.CompilerParams(use_tc_tiling_on_sc=False),
        out_type=(
            jax.ShapeDtypeStruct((NC, NUM_GRAPHS, D_V), jnp.float32),
            jax.ShapeDtypeStruct((NC, NUM_GRAPHS, D_E), jnp.float32),
        ),
        scratch_types=(
            pltpu.VMEM((CV, D_V), jnp.float32),
            pltpu.VMEM((CV,), jnp.int32),
            pltpu.VMEM((NV_SUB, CV_SUB), jnp.int32),
            pltpu.VMEM((NUM_GRAPHS, D_E), jnp.float32),
            pltpu.VMEM((CV_SUB, D_E), jnp.float32),
            pltpu.VMEM_SHARED((NUM_GRAPHS, D_V), jnp.float32),
            pltpu.VMEM_SHARED((NUM_GRAPHS, D_E), jnp.float32),
        ),
    )
    def k(vdata_h, vidx_h, vout, cout,
          vbuf, vflat, vibuf, zbuf, ones, v_acc, c_acc):
        c = lax.axis_index("c")
        s = lax.axis_index("s")
        wid = s * NC + c

        one = jnp.full((16,), 1.0, jnp.float32)
        zero = jnp.zeros((16,), jnp.float32)

        def ones_row(r, carry):
            ones[r, pl.ds(0, 16)] = one
            return carry

        lax.fori_loop(0, CV_SUB, ones_row, 0)

        @pl.when(s == 0)
        def _():
            def zv(r, carry):
                for j in range(D_V // 16):
                    vbuf[r, pl.ds(j * 16, 16)] = zero
                return carry

            lax.fori_loop(0, NUM_GRAPHS, zv, 0)

            def ze(r, carry):
                zbuf[r, pl.ds(0, 16)] = zero
                return carry

            lax.fori_loop(0, NUM_GRAPHS, ze, 0)
            pltpu.sync_copy(vbuf.at[pl.ds(0, NUM_GRAPHS)], v_acc)
            pltpu.sync_copy(zbuf, c_acc)

        plsc.subcore_barrier()

        def vchunk(kk, carry):
            chunk = wid + NW * kk

            @pl.when(chunk < NV_CHUNKS)
            def _():
                pltpu.sync_copy(vdata_h.at[pl.ds(chunk * CV, CV)], vbuf)
                pltpu.sync_copy(vidx_h.at[pl.ds(chunk * CV, CV)], vflat)
                for j in range(NV_SUB):
                    for l in range(CV_SUB // 16):
                        vibuf[j, pl.ds(l * 16, 16)] = vflat[
                            pl.ds(j * CV_SUB + l * 16, 16)]
                for j in range(NV_SUB):
                    idx = vibuf.at[j]
                    pltpu.sync_copy(
                        vbuf.at[pl.ds(j * CV_SUB, CV_SUB)],
                        v_acc.at[idx], add=True)
                    pltpu.sync_copy(ones, c_acc.at[idx], add=True)

            return carry

        lax.fori_loop(0, NV_ITERS, vchunk, 0)

        plsc.subcore_barrier()

        @pl.when(s == 0)
        def _():
            pltpu.sync_copy(v_acc, vout.at[c])
            pltpu.sync_copy(c_acc, cout.at[c])

    return k(vdata, vidx)


def _tc_edge_full(edata_t, seg_first, seg_last):
    def body(first_ref, last_ref, e_ref, acc_ref):
        bi = pl.program_id(0)

        @pl.when(bi == 0)
        def _():
            acc_ref[...] = jnp.zeros_like(acc_ref)

        iota_row = lax.broadcasted_iota(jnp.int32, (1, NUM_GRAPHS), 1)
        upd = jnp.zeros((D_E, NUM_GRAPHS), jnp.float32)
        for u in range(U):
            ci = bi * U + u
            s0 = first_ref[ci]
            s1 = last_ref[ci]
            blk = e_ref[:, u * KE:(u + 1) * KE]
            csum = jnp.sum(blk, axis=1, keepdims=True)
            sel = jnp.where(s0 == s1, (iota_row == s0).astype(jnp.float32),
                            jnp.zeros((1, NUM_GRAPHS), jnp.float32))
            upd = upd + csum * sel
        acc_ref[...] += upd

    grid_spec = pltpu.PrefetchScalarGridSpec(
        num_scalar_prefetch=2,
        grid=(NB,),
        in_specs=[
            pl.BlockSpec((D_E, KB), lambda bi, *_: (0, bi)),
        ],
        out_specs=pl.BlockSpec((D_E, NUM_GRAPHS), lambda bi, *_: (0, 0)),
    )
    return pl.pallas_call(
        body,
        grid_spec=grid_spec,
        out_shape=jax.ShapeDtypeStruct((D_E, NUM_GRAPHS), jnp.float32),
    )(seg_first, seg_last, edata_t)


def _tc_edge_mixed(edata_t, eidx3, mixed_ids, n_mixed):
    def body(mid_ref, n_ref, e_ref, idx_ref, acc_ref, m_ref):
        i = pl.program_id(0)

        @pl.when(i == 0)
        def _():
            acc_ref[...] = jnp.zeros_like(acc_ref)

        valid = i < n_ref[0]
        iota = lax.broadcasted_iota(jnp.int32, (NUM_GRAPHS, 1), 0)
        for j in range(KROWS):
            idrow = idx_ref[:, j, :]
            m = jnp.where(valid, (iota == idrow).astype(jnp.float32),
                          jnp.zeros((NUM_GRAPHS, 128), jnp.float32))
            m_ref[:, j * 128:(j + 1) * 128] = m
        upd = lax.dot_general(
            e_ref[...], m_ref[...], (((1,), (1,)), ((), ())),
            precision=lax.Precision.HIGHEST,
            preferred_element_type=jnp.float32)
        acc_ref[...] += upd

    grid_spec = pltpu.PrefetchScalarGridSpec(
        num_scalar_prefetch=2,
        grid=(MAXM,),
        in_specs=[
            pl.BlockSpec((D_E, KE), lambda i, mid, n: (0, mid[i])),
            pl.BlockSpec((1, KROWS, 128), lambda i, mid, n: (mid[i], 0, 0)),
        ],
        out_specs=pl.BlockSpec((D_E, NUM_GRAPHS), lambda i, *_: (0, 0)),
        scratch_shapes=[pltpu.VMEM((NUM_GRAPHS, KE), jnp.float32)],
    )
    return pl.pallas_call(
        body,
        grid_spec=grid_spec,
        out_shape=jax.ShapeDtypeStruct((D_E, NUM_GRAPHS), jnp.float32),
    )(mixed_ids, n_mixed, edata_t, eidx3)


def _tc_final(cdata, vpart, cpart, e_full, e_mix, W, b2):
    def body(c_ref, v_ref, cnt_ref, ef_ref, em_ref, w_ref, b_ref, o_ref):
        v_sum = v_ref[0] + v_ref[1]
        cnt = cnt_ref[0] + cnt_ref[1]
        denom = jnp.maximum(jnp.max(cnt, axis=1, keepdims=True), 1.0)
        v_agg = v_sum / denom
        e_agg_t = ef_ref[...] + em_ref[...]
        out = (
            jnp.dot(c_ref[...], w_ref[0:D_V, :],
                    preferred_element_type=jnp.float32)
            + jnp.dot(v_agg, w_ref[D_V:2 * D_V, :],
                      preferred_element_type=jnp.float32)
            + lax.dot_general(
                e_agg_t, w_ref[2 * D_V:2 * D_V + D_E, :],
                (((0,), (0,)), ((), ())),
                preferred_element_type=jnp.float32)
            + b_ref[...]
        )
        o_ref[...] = out

    return pl.pallas_call(
        body,
        out_shape=jax.ShapeDtypeStruct((NUM_GRAPHS, 128), jnp.float32),
    )(cdata, vpart, cpart, e_full, e_mix, W, b2)


def kernel(cdata, vdata, edata_e, vidx, eidx, W, b):
    edata_t = edata_e.T
    seg_first = eidx[::KE].astype(jnp.int32)
    seg_last = eidx[KE - 1::KE].astype(jnp.int32)
    mixed = seg_first != seg_last
    mixed_ids = jnp.nonzero(mixed, size=MAXM, fill_value=0)[0].astype(
        jnp.int32)
    n_mixed = jnp.sum(mixed.astype(jnp.int32)).reshape(1)
    eidx3 = eidx.reshape(NE_CHUNKS, KROWS, 128)

    vp, cp = _sc_vertex_sums(vdata, vidx)
    e_full = _tc_edge_full(edata_t, seg_first, seg_last)
    e_mix = _tc_edge_mixed(edata_t, eidx3, mixed_ids, n_mixed)
    return _tc_final(cdata, vp, cp, e_full, e_mix, W, b.reshape(1, -1))


# speedup vs baseline: 27.9642x; 1.3272x over previous
import functools

import jax
import jax.numpy as jnp
from jax import lax
from jax.experimental import pallas as pl
from jax.experimental.pallas import tpu as pltpu
from jax.experimental.pallas import tpu_sc as plsc

NUM_GRAPHS = 100
N_VERT = 50000
N_EDGE = 1600000
D_V = 128
D_E = 16

NC, NS = 2, 16
NW = NC * NS

CV = 400
NV_CHUNKS = N_VERT // CV
CV_SUB = 80
NV_SUB = CV // CV_SUB
NV_ITERS = -(-NV_CHUNKS // NW)

KE = 2560
NE_CHUNKS = N_EDGE // KE
U = 25
KB = KE * U
NB = NE_CHUNKS // U
KROWS = KE // 128
MAXM = 100
MP = 2


def _sc_vertex_sums(vdata, vidx):
    mesh = plsc.VectorSubcoreMesh(core_axis_name="c", subcore_axis_name="s")

    @functools.partial(
        pl.kernel,
        mesh=mesh,
        compiler_params=pltpu.CompilerParams(use_tc_tiling_on_sc=False),
        out_type=(
            jax.ShapeDtypeStruct((NC, NUM_GRAPHS, D_V), jnp.float32),
            jax.ShapeDtypeStruct((NC, NUM_GRAPHS, D_E), jnp.float32),
        ),
        scratch_types=(
            pltpu.VMEM((CV, D_V), jnp.float32),
            pltpu.VMEM((CV,), jnp.int32),
            pltpu.VMEM((NV_SUB, CV_SUB), jnp.int32),
            pltpu.VMEM((NUM_GRAPHS, D_E), jnp.float32),
            pltpu.VMEM((CV_SUB, D_E), jnp.float32),
            pltpu.VMEM_SHARED((NUM_GRAPHS, D_V), jnp.float32),
            pltpu.VMEM_SHARED((NUM_GRAPHS, D_E), jnp.float32),
        ),
    )
    def k(vdata_h, vidx_h, vout, cout,
          vbuf, vflat, vibuf, zbuf, ones, v_acc, c_acc):
        c = lax.axis_index("c")
        s = lax.axis_index("s")
        wid = s * NC + c

        one = jnp.full((16,), 1.0, jnp.float32)
        zero = jnp.zeros((16,), jnp.float32)

        def ones_row(r, carry):
            ones[r, pl.ds(0, 16)] = one
            return carry

        lax.fori_loop(0, CV_SUB, ones_row, 0)

        @pl.when(s == 0)
        def _():
            def zv(r, carry):
                for j in range(D_V // 16):
                    vbuf[r, pl.ds(j * 16, 16)] = zero
                return carry

            lax.fori_loop(0, NUM_GRAPHS, zv, 0)

            def ze(r, carry):
                zbuf[r, pl.ds(0, 16)] = zero
                return carry

            lax.fori_loop(0, NUM_GRAPHS, ze, 0)
            pltpu.sync_copy(vbuf.at[pl.ds(0, NUM_GRAPHS)], v_acc)
            pltpu.sync_copy(zbuf, c_acc)

        plsc.subcore_barrier()

        def vchunk(kk, carry):
            chunk = wid + NW * kk

            @pl.when(chunk < NV_CHUNKS)
            def _():
                pltpu.sync_copy(vdata_h.at[pl.ds(chunk * CV, CV)], vbuf)
                pltpu.sync_copy(vidx_h.at[pl.ds(chunk * CV, CV)], vflat)
                for j in range(NV_SUB):
                    for l in range(CV_SUB // 16):
                        vibuf[j, pl.ds(l * 16, 16)] = vflat[
                            pl.ds(j * CV_SUB + l * 16, 16)]
                for j in range(NV_SUB):
                    idx = vibuf.at[j]
                    pltpu.sync_copy(
                        vbuf.at[pl.ds(j * CV_SUB, CV_SUB)],
                        v_acc.at[idx], add=True)
                    pltpu.sync_copy(ones, c_acc.at[idx], add=True)

            return carry

        lax.fori_loop(0, NV_ITERS, vchunk, 0)

        plsc.subcore_barrier()

        @pl.when(s == 0)
        def _():
            pltpu.sync_copy(v_acc, vout.at[c])
            pltpu.sync_copy(c_acc, cout.at[c])

    return k(vdata, vidx)


def _tc_edge_full(edata_t, seg_first, seg_last):
    def body(first_ref, last_ref, e_ref, acc_ref):
        bi = pl.program_id(0)

        @pl.when(bi == 0)
        def _():
            acc_ref[...] = jnp.zeros_like(acc_ref)

        iota_row = lax.broadcasted_iota(jnp.int32, (1, NUM_GRAPHS), 1)
        upd = jnp.zeros((D_E, NUM_GRAPHS), jnp.float32)
        for u in range(U):
            ci = bi * U + u
            s0 = first_ref[ci]
            s1 = last_ref[ci]
            blk = e_ref[:, u * KE:(u + 1) * KE]
            csum = jnp.sum(blk, axis=1, keepdims=True)
            sel = jnp.where(s0 == s1, (iota_row == s0).astype(jnp.float32),
                            jnp.zeros((1, NUM_GRAPHS), jnp.float32))
            upd = upd + csum * sel
        acc_ref[...] += upd

    grid_spec = pltpu.PrefetchScalarGridSpec(
        num_scalar_prefetch=2,
        grid=(NB,),
        in_specs=[
            pl.BlockSpec((D_E, KB), lambda bi, *_: (0, bi)),
        ],
        out_specs=pl.BlockSpec((D_E, NUM_GRAPHS), lambda bi, *_: (0, 0)),
    )
    return pl.pallas_call(
        body,
        grid_spec=grid_spec,
        out_shape=jax.ShapeDtypeStruct((D_E, NUM_GRAPHS), jnp.float32),
    )(seg_first, seg_last, edata_t)


def _tc_edge_mixed(edata_t, eidx3, mixed_ids, n_mixed):
    def body(mid_ref, n_ref, e_ref_a, e_ref_b, idx_ref_a, idx_ref_b,
             acc_ref, m_ref):
        i = pl.program_id(0)

        @pl.when(i == 0)
        def _():
            acc_ref[...] = jnp.zeros_like(acc_ref)

        iota = lax.broadcasted_iota(jnp.int32, (NUM_GRAPHS, 1), 0)
        upd = jnp.zeros((D_E, NUM_GRAPHS), jnp.float32)
        for h, (e_ref, idx_ref) in enumerate(
                ((e_ref_a, idx_ref_a), (e_ref_b, idx_ref_b))):
            valid = (MP * i + h) < n_ref[0]
            for j in range(KROWS):
                idrow = idx_ref[:, j, :]
                m = jnp.where(valid, (iota == idrow).astype(jnp.float32),
                              jnp.zeros((NUM_GRAPHS, 128), jnp.float32))
                m_ref[:, j * 128:(j + 1) * 128] = m
            upd = upd + lax.dot_general(
                e_ref[...], m_ref[...], (((1,), (1,)), ((), ())),
                precision=lax.Precision.HIGHEST,
                preferred_element_type=jnp.float32)
        acc_ref[...] += upd

    grid_spec = pltpu.PrefetchScalarGridSpec(
        num_scalar_prefetch=2,
        grid=(MAXM // MP,),
        in_specs=[
            pl.BlockSpec((D_E, KE), lambda i, mid, n: (0, mid[MP * i])),
            pl.BlockSpec((D_E, KE), lambda i, mid, n: (0, mid[MP * i + 1])),
            pl.BlockSpec((1, KROWS, 128),
                         lambda i, mid, n: (mid[MP * i], 0, 0)),
            pl.BlockSpec((1, KROWS, 128),
                         lambda i, mid, n: (mid[MP * i + 1], 0, 0)),
        ],
        out_specs=pl.BlockSpec((D_E, NUM_GRAPHS), lambda i, *_: (0, 0)),
        scratch_shapes=[pltpu.VMEM((NUM_GRAPHS, KE), jnp.float32)],
    )
    return pl.pallas_call(
        body,
        grid_spec=grid_spec,
        out_shape=jax.ShapeDtypeStruct((D_E, NUM_GRAPHS), jnp.float32),
    )(mixed_ids, n_mixed, edata_t, edata_t, eidx3, eidx3)


def _tc_final(cdata, vpart, cpart, e_full, e_mix, W, b2):
    def body(c_ref, v_ref, cnt_ref, ef_ref, em_ref, w_ref, b_ref, o_ref):
        v_sum = v_ref[0] + v_ref[1]
        cnt = cnt_ref[0] + cnt_ref[1]
        denom = jnp.maximum(jnp.max(cnt, axis=1, keepdims=True), 1.0)
        v_agg = v_sum / denom
        e_agg_t = ef_ref[...] + em_ref[...]
        out = (
            jnp.dot(c_ref[...], w_ref[0:D_V, :],
                    preferred_element_type=jnp.float32)
            + jnp.dot(v_agg, w_ref[D_V:2 * D_V, :],
                      preferred_element_type=jnp.float32)
            + lax.dot_general(
                e_agg_t, w_ref[2 * D_V:2 * D_V + D_E, :],
                (((0,), (0,)), ((), ())),
                preferred_element_type=jnp.float32)
            + b_ref[...]
        )
        o_ref[...] = out

    return pl.pallas_call(
        body,
        out_shape=jax.ShapeDtypeStruct((NUM_GRAPHS, 128), jnp.float32),
    )(cdata, vpart, cpart, e_full, e_mix, W, b2)


def kernel(cdata, vdata, edata_e, vidx, eidx, W, b):
    edata_t = edata_e.T
    seg_first = eidx[::KE].astype(jnp.int32)
    seg_last = eidx[KE - 1::KE].astype(jnp.int32)
    mixed = seg_first != seg_last
    mixed_ids = jnp.nonzero(mixed, size=MAXM, fill_value=0)[0].astype(
        jnp.int32)
    n_mixed = jnp.sum(mixed.astype(jnp.int32)).reshape(1)
    eidx3 = eidx.reshape(NE_CHUNKS, KROWS, 128)

    vp, cp = _sc_vertex_sums(vdata, vidx)
    e_full = _tc_edge_full(edata_t, seg_first, seg_last)
    e_mix = _tc_edge_mixed(edata_t, eidx3, mixed_ids, n_mixed)
    return _tc_final(cdata, vp, cp, e_full, e_mix, W, b.reshape(1, -1))


# speedup vs baseline: 38.0146x; 1.3594x over previous
import functools

import jax
import jax.numpy as jnp
from jax import lax
from jax.experimental import pallas as pl
from jax.experimental.pallas import tpu as pltpu
from jax.experimental.pallas import tpu_sc as plsc

NUM_GRAPHS = 100
N_VERT = 50000
N_EDGE = 1600000
D_V = 128
D_E = 16

NC, NS = 2, 16
NW = NC * NS

CV = 400
NV_CHUNKS = N_VERT // CV
CV_SUB = 80
NV_SUB = CV // CV_SUB
NV_ITERS = -(-NV_CHUNKS // NW)

KE = 640
NE_CHUNKS = N_EDGE // KE
U = 100
KB = KE * U
NB = NE_CHUNKS // U
KROWS = KE // 128
MAXM = 100
MP = 2


def _sc_vertex_sums(vdata, vidx):
    mesh = plsc.VectorSubcoreMesh(core_axis_name="c", subcore_axis_name="s")

    @functools.partial(
        pl.kernel,
        mesh=mesh,
        compiler_params=pltpu.CompilerParams(use_tc_tiling_on_sc=False),
        out_type=(
            jax.ShapeDtypeStruct((NC, NUM_GRAPHS, D_V), jnp.float32),
            jax.ShapeDtypeStruct((NC, NUM_GRAPHS, D_E), jnp.float32),
        ),
        scratch_types=(
            pltpu.VMEM((CV, D_V), jnp.float32),
            pltpu.VMEM((CV,), jnp.int32),
            pltpu.VMEM((NV_SUB, CV_SUB), jnp.int32),
            pltpu.VMEM((NUM_GRAPHS, D_E), jnp.float32),
            pltpu.VMEM((CV_SUB, D_E), jnp.float32),
            pltpu.VMEM_SHARED((NUM_GRAPHS, D_V), jnp.float32),
            pltpu.VMEM_SHARED((NUM_GRAPHS, D_E), jnp.float32),
        ),
    )
    def k(vdata_h, vidx_h, vout, cout,
          vbuf, vflat, vibuf, zbuf, ones, v_acc, c_acc):
        c = lax.axis_index("c")
        s = lax.axis_index("s")
        wid = s * NC + c

        one = jnp.full((16,), 1.0, jnp.float32)
        zero = jnp.zeros((16,), jnp.float32)

        def ones_row(r, carry):
            ones[r, pl.ds(0, 16)] = one
            return carry

        lax.fori_loop(0, CV_SUB, ones_row, 0)

        @pl.when(s == 0)
        def _():
            def zv(r, carry):
                for j in range(D_V // 16):
                    vbuf[r, pl.ds(j * 16, 16)] = zero
                return carry

            lax.fori_loop(0, NUM_GRAPHS, zv, 0)

            def ze(r, carry):
                zbuf[r, pl.ds(0, 16)] = zero
                return carry

            lax.fori_loop(0, NUM_GRAPHS, ze, 0)
            pltpu.sync_copy(vbuf.at[pl.ds(0, NUM_GRAPHS)], v_acc)
            pltpu.sync_copy(zbuf, c_acc)

        plsc.subcore_barrier()

        def vchunk(kk, carry):
            chunk = wid + NW * kk

            @pl.when(chunk < NV_CHUNKS)
            def _():
                pltpu.sync_copy(vdata_h.at[pl.ds(chunk * CV, CV)], vbuf)
                pltpu.sync_copy(vidx_h.at[pl.ds(chunk * CV, CV)], vflat)
                for j in range(NV_SUB):
                    for l in range(CV_SUB // 16):
                        vibuf[j, pl.ds(l * 16, 16)] = vflat[
                            pl.ds(j * CV_SUB + l * 16, 16)]
                for j in range(NV_SUB):
                    idx = vibuf.at[j]
                    pltpu.sync_copy(
                        vbuf.at[pl.ds(j * CV_SUB, CV_SUB)],
                        v_acc.at[idx], add=True)
                    pltpu.sync_copy(ones, c_acc.at[idx], add=True)

            return carry

        lax.fori_loop(0, NV_ITERS, vchunk, 0)

        plsc.subcore_barrier()

        @pl.when(s == 0)
        def _():
            pltpu.sync_copy(v_acc, vout.at[c])
            pltpu.sync_copy(c_acc, cout.at[c])

    return k(vdata, vidx)


def _tc_edge_full(edata_t, seg_first, seg_last):
    def body(first_ref, last_ref, e_ref, acc_ref):
        bi = pl.program_id(0)

        @pl.when(bi == 0)
        def _():
            acc_ref[...] = jnp.zeros_like(acc_ref)

        iota_row = lax.broadcasted_iota(jnp.int32, (1, NUM_GRAPHS), 1)
        upd = jnp.zeros((D_E, NUM_GRAPHS), jnp.float32)
        for u in range(U):
            ci = bi * U + u
            s0 = first_ref[ci]
            s1 = last_ref[ci]
            blk = e_ref[:, u * KE:(u + 1) * KE]
            csum = jnp.sum(blk, axis=1, keepdims=True)
            sel = jnp.where(s0 == s1, (iota_row == s0).astype(jnp.float32),
                            jnp.zeros((1, NUM_GRAPHS), jnp.float32))
            upd = upd + csum * sel
        acc_ref[...] += upd

    grid_spec = pltpu.PrefetchScalarGridSpec(
        num_scalar_prefetch=2,
        grid=(NB,),
        in_specs=[
            pl.BlockSpec((D_E, KB), lambda bi, *_: (0, bi)),
        ],
        out_specs=pl.BlockSpec((D_E, NUM_GRAPHS), lambda bi, *_: (0, 0)),
    )
    return pl.pallas_call(
        body,
        grid_spec=grid_spec,
        out_shape=jax.ShapeDtypeStruct((D_E, NUM_GRAPHS), jnp.float32),
    )(seg_first, seg_last, edata_t)


def _tc_edge_mixed(edata_t, eidx3, mixed_ids, n_mixed):
    def body(mid_ref, n_ref, e_ref_a, e_ref_b, idx_ref_a, idx_ref_b,
             acc_ref, m_ref):
        i = pl.program_id(0)

        @pl.when(i == 0)
        def _():
            acc_ref[...] = jnp.zeros_like(acc_ref)

        iota = lax.broadcasted_iota(jnp.int32, (NUM_GRAPHS, 1), 0)
        upd = jnp.zeros((D_E, NUM_GRAPHS), jnp.float32)
        for h, (e_ref, idx_ref) in enumerate(
                ((e_ref_a, idx_ref_a), (e_ref_b, idx_ref_b))):
            valid = (MP * i + h) < n_ref[0]
            for j in range(KROWS):
                idrow = idx_ref[:, j, :]
                m = jnp.logical_and(iota == idrow, valid)
                m_ref[:, j * 128:(j + 1) * 128] = m.astype(jnp.bfloat16)
            blk = e_ref[...]
            bhi = blk.astype(jnp.bfloat16)
            blo = (blk - bhi.astype(jnp.float32)).astype(jnp.bfloat16)
            mm = m_ref[...]
            upd = upd + lax.dot_general(
                bhi, mm, (((1,), (1,)), ((), ())),
                preferred_element_type=jnp.float32)
            upd = upd + lax.dot_general(
                blo, mm, (((1,), (1,)), ((), ())),
                preferred_element_type=jnp.float32)
        acc_ref[...] += upd

    grid_spec = pltpu.PrefetchScalarGridSpec(
        num_scalar_prefetch=2,
        grid=(MAXM // MP,),
        in_specs=[
            pl.BlockSpec((D_E, KE), lambda i, mid, n: (0, mid[MP * i])),
            pl.BlockSpec((D_E, KE), lambda i, mid, n: (0, mid[MP * i + 1])),
            pl.BlockSpec((1, KROWS, 128),
                         lambda i, mid, n: (mid[MP * i], 0, 0)),
            pl.BlockSpec((1, KROWS, 128),
                         lambda i, mid, n: (mid[MP * i + 1], 0, 0)),
        ],
        out_specs=pl.BlockSpec((D_E, NUM_GRAPHS), lambda i, *_: (0, 0)),
        scratch_shapes=[pltpu.VMEM((NUM_GRAPHS, KE), jnp.bfloat16)],
    )
    return pl.pallas_call(
        body,
        grid_spec=grid_spec,
        out_shape=jax.ShapeDtypeStruct((D_E, NUM_GRAPHS), jnp.float32),
    )(mixed_ids, n_mixed, edata_t, edata_t, eidx3, eidx3)


def _tc_final(cdata, vpart, cpart, e_full, e_mix, W, b2):
    def body(c_ref, v_ref, cnt_ref, ef_ref, em_ref, w_ref, b_ref, o_ref):
        v_sum = v_ref[0] + v_ref[1]
        cnt = cnt_ref[0] + cnt_ref[1]
        denom = jnp.maximum(jnp.max(cnt, axis=1, keepdims=True), 1.0)
        v_agg = v_sum / denom
        e_agg_t = ef_ref[...] + em_ref[...]
        out = (
            jnp.dot(c_ref[...], w_ref[0:D_V, :],
                    preferred_element_type=jnp.float32)
            + jnp.dot(v_agg, w_ref[D_V:2 * D_V, :],
                      preferred_element_type=jnp.float32)
            + lax.dot_general(
                e_agg_t, w_ref[2 * D_V:2 * D_V + D_E, :],
                (((0,), (0,)), ((), ())),
                preferred_element_type=jnp.float32)
            + b_ref[...]
        )
        o_ref[...] = out

    return pl.pallas_call(
        body,
        out_shape=jax.ShapeDtypeStruct((NUM_GRAPHS, 128), jnp.float32),
    )(cdata, vpart, cpart, e_full, e_mix, W, b2)


def kernel(cdata, vdata, edata_e, vidx, eidx, W, b):
    edata_t = edata_e.T
    seg_first = eidx[::KE].astype(jnp.int32)
    seg_last = eidx[KE - 1::KE].astype(jnp.int32)
    mixed = seg_first != seg_last
    mixed_ids = jnp.nonzero(mixed, size=MAXM, fill_value=0)[0].astype(
        jnp.int32)
    n_mixed = jnp.sum(mixed.astype(jnp.int32)).reshape(1)
    eidx3 = eidx.reshape(NE_CHUNKS, KROWS, 128)

    vp, cp = _sc_vertex_sums(vdata, vidx)
    e_full = _tc_edge_full(edata_t, seg_first, seg_last)
    e_mix = _tc_edge_mixed(edata_t, eidx3, mixed_ids, n_mixed)
    return _tc_final(cdata, vp, cp, e_full, e_mix, W, b.reshape(1, -1))


# speedup vs baseline: 48.9803x; 1.2885x over previous
import functools

import jax
import jax.numpy as jnp
from jax import lax
from jax.experimental import pallas as pl
from jax.experimental.pallas import tpu as pltpu
from jax.experimental.pallas import tpu_sc as plsc

NUM_GRAPHS = 100
N_VERT = 50000
N_EDGE = 1600000
D_V = 128
D_E = 16

NC, NS = 2, 16
NW = NC * NS

CV = 400
NV_CHUNKS = N_VERT // CV
CV_SUB = 80
NV_SUB = CV // CV_SUB
NV_ITERS = -(-NV_CHUNKS // NW)

KE = 640
NE_CHUNKS = N_EDGE // KE
U = 100
KB = KE * U
NB = NE_CHUNKS // U
KROWS = KE // 128
MAXM = 100
MP = 4


def _sc_vertex_sums(vdata, vidx):
    mesh = plsc.VectorSubcoreMesh(core_axis_name="c", subcore_axis_name="s")

    @functools.partial(
        pl.kernel,
        mesh=mesh,
        compiler_params=pltpu.CompilerParams(use_tc_tiling_on_sc=False),
        out_type=(
            jax.ShapeDtypeStruct((NC, NUM_GRAPHS, D_V), jnp.float32),
            jax.ShapeDtypeStruct((NC, NUM_GRAPHS, D_E), jnp.float32),
        ),
        scratch_types=(
            pltpu.VMEM((CV, D_V), jnp.float32),
            pltpu.VMEM((CV,), jnp.int32),
            pltpu.VMEM((NV_SUB, CV_SUB), jnp.int32),
            pltpu.VMEM((NUM_GRAPHS, D_E), jnp.float32),
            pltpu.VMEM((CV_SUB, D_E), jnp.float32),
            pltpu.VMEM_SHARED((NUM_GRAPHS, D_V), jnp.float32),
            pltpu.VMEM_SHARED((NUM_GRAPHS, D_E), jnp.float32),
        ),
    )
    def k(vdata_h, vidx_h, vout, cout,
          vbuf, vflat, vibuf, zbuf, ones, v_acc, c_acc):
        c = lax.axis_index("c")
        s = lax.axis_index("s")
        wid = s * NC + c

        one = jnp.full((16,), 1.0, jnp.float32)
        zero = jnp.zeros((16,), jnp.float32)

        def ones_row(r, carry):
            ones[r, pl.ds(0, 16)] = one
            return carry

        lax.fori_loop(0, CV_SUB, ones_row, 0)

        @pl.when(s == 0)
        def _():
            def zv(r, carry):
                for j in range(D_V // 16):
                    vbuf[r, pl.ds(j * 16, 16)] = zero
                return carry

            lax.fori_loop(0, NUM_GRAPHS, zv, 0)

            def ze(r, carry):
                zbuf[r, pl.ds(0, 16)] = zero
                return carry

            lax.fori_loop(0, NUM_GRAPHS, ze, 0)
            pltpu.sync_copy(vbuf.at[pl.ds(0, NUM_GRAPHS)], v_acc)
            pltpu.sync_copy(zbuf, c_acc)

        plsc.subcore_barrier()

        def vchunk(kk, carry):
            chunk = wid + NW * kk

            @pl.when(chunk < NV_CHUNKS)
            def _():
                pltpu.sync_copy(vdata_h.at[pl.ds(chunk * CV, CV)], vbuf)
                pltpu.sync_copy(vidx_h.at[pl.ds(chunk * CV, CV)], vflat)
                for j in range(NV_SUB):
                    for l in range(CV_SUB // 16):
                        vibuf[j, pl.ds(l * 16, 16)] = vflat[
                            pl.ds(j * CV_SUB + l * 16, 16)]
                for j in range(NV_SUB):
                    idx = vibuf.at[j]
                    pltpu.sync_copy(
                        vbuf.at[pl.ds(j * CV_SUB, CV_SUB)],
                        v_acc.at[idx], add=True)
                    pltpu.sync_copy(ones, c_acc.at[idx], add=True)

            return carry

        lax.fori_loop(0, NV_ITERS, vchunk, 0)

        plsc.subcore_barrier()

        @pl.when(s == 0)
        def _():
            pltpu.sync_copy(v_acc, vout.at[c])
            pltpu.sync_copy(c_acc, cout.at[c])

    return k(vdata, vidx)


def _tc_edge_full(edata_t, seg_first, seg_last):
    def body(first_ref, last_ref, e_ref, acc_ref):
        bi = pl.program_id(0)

        @pl.when(bi == 0)
        def _():
            acc_ref[...] = jnp.zeros_like(acc_ref)

        iota_row = lax.broadcasted_iota(jnp.int32, (1, NUM_GRAPHS), 1)
        upd = jnp.zeros((D_E, NUM_GRAPHS), jnp.float32)
        for u in range(U):
            ci = bi * U + u
            s0 = first_ref[ci]
            s1 = last_ref[ci]
            blk = e_ref[:, u * KE:(u + 1) * KE]
            csum = jnp.sum(blk, axis=1, keepdims=True)
            sel = jnp.where(s0 == s1, (iota_row == s0).astype(jnp.float32),
                            jnp.zeros((1, NUM_GRAPHS), jnp.float32))
            upd = upd + csum * sel
        acc_ref[...] += upd

    grid_spec = pltpu.PrefetchScalarGridSpec(
        num_scalar_prefetch=2,
        grid=(NB,),
        in_specs=[
            pl.BlockSpec((D_E, KB), lambda bi, *_: (0, bi)),
        ],
        out_specs=pl.BlockSpec((D_E, NUM_GRAPHS), lambda bi, *_: (0, 0)),
    )
    return pl.pallas_call(
        body,
        grid_spec=grid_spec,
        out_shape=jax.ShapeDtypeStruct((D_E, NUM_GRAPHS), jnp.float32),
    )(seg_first, seg_last, edata_t)


def _tc_edge_mixed(edata_t, eidx3, mixed_ids, valid_arr):
    def body(mid_ref, val_ref, *refs):
        e_refs = refs[0:MP]
        idx_refs = refs[MP:2 * MP]
        acc_ref = refs[2 * MP]
        m_ref = refs[2 * MP + 1]
        i = pl.program_id(0)

        @pl.when(i == 0)
        def _():
            acc_ref[...] = jnp.zeros_like(acc_ref)

        iota = lax.broadcasted_iota(jnp.int32, (NUM_GRAPHS, 1), 0)
        upd = jnp.zeros((D_E, NUM_GRAPHS), jnp.float32)
        for h in range(MP):
            e_ref = e_refs[h]
            idx_ref = idx_refs[h]
            valid = val_ref[MP * i + h] != 0
            for j in range(KROWS):
                idrow = idx_ref[:, j, :]
                m = jnp.logical_and(iota == idrow, valid)
                m_ref[:, j * 128:(j + 1) * 128] = m.astype(jnp.bfloat16)
            blk = e_ref[...]
            bhi = blk.astype(jnp.bfloat16)
            blo = (blk - bhi.astype(jnp.float32)).astype(jnp.bfloat16)
            mm = m_ref[...]
            upd = upd + lax.dot_general(
                bhi, mm, (((1,), (1,)), ((), ())),
                preferred_element_type=jnp.float32)
            upd = upd + lax.dot_general(
                blo, mm, (((1,), (1,)), ((), ())),
                preferred_element_type=jnp.float32)
        acc_ref[...] += upd

    def e_spec(h):
        return pl.BlockSpec((D_E, KE), lambda i, mid, val: (0, mid[MP * i + h]))

    def i_spec(h):
        return pl.BlockSpec((1, KROWS, 128),
                            lambda i, mid, val: (mid[MP * i + h], 0, 0))

    grid_spec = pltpu.PrefetchScalarGridSpec(
        num_scalar_prefetch=2,
        grid=(MAXM // MP,),
        in_specs=[e_spec(h) for h in range(MP)]
        + [i_spec(h) for h in range(MP)],
        out_specs=pl.BlockSpec((D_E, NUM_GRAPHS), lambda i, *_: (0, 0)),
        scratch_shapes=[pltpu.VMEM((NUM_GRAPHS, KE), jnp.bfloat16)],
    )
    return pl.pallas_call(
        body,
        grid_spec=grid_spec,
        out_shape=jax.ShapeDtypeStruct((D_E, NUM_GRAPHS), jnp.float32),
    )(mixed_ids, valid_arr, *([edata_t] * MP), *([eidx3] * MP))


def _tc_final(cdata, vpart, cpart, e_full, e_mix, W, b2):
    def body(c_ref, v_ref, cnt_ref, ef_ref, em_ref, w_ref, b_ref, o_ref):
        v_sum = v_ref[0] + v_ref[1]
        cnt = cnt_ref[0] + cnt_ref[1]
        denom = jnp.maximum(jnp.max(cnt, axis=1, keepdims=True), 1.0)
        v_agg = v_sum / denom
        e_agg_t = ef_ref[...] + em_ref[...]
        out = (
            jnp.dot(c_ref[...], w_ref[0:D_V, :],
                    preferred_element_type=jnp.float32)
            + jnp.dot(v_agg, w_ref[D_V:2 * D_V, :],
                      preferred_element_type=jnp.float32)
            + lax.dot_general(
                e_agg_t, w_ref[2 * D_V:2 * D_V + D_E, :],
                (((0,), (0,)), ((), ())),
                preferred_element_type=jnp.float32)
            + b_ref[...]
        )
        o_ref[...] = out

    return pl.pallas_call(
        body,
        out_shape=jax.ShapeDtypeStruct((NUM_GRAPHS, 128), jnp.float32),
    )(cdata, vpart, cpart, e_full, e_mix, W, b2)


def kernel(cdata, vdata, edata_e, vidx, eidx, W, b):
    edata_t = edata_e.T
    seg_first = eidx[::KE].astype(jnp.int32)
    seg_last = eidx[KE - 1::KE].astype(jnp.int32)
    svals = jnp.arange(1, MAXM + 1, dtype=jnp.int32)
    cnts = jnp.sum((seg_last[None, :] < svals[:, None]).astype(jnp.int32),
                   axis=1)
    ids = jnp.minimum(cnts, NE_CHUNKS - 1)
    prev = jnp.concatenate([jnp.full((1,), -1, jnp.int32), ids[:-1]])
    valid_arr = ((seg_first[ids] != seg_last[ids])
                 & (ids != prev)).astype(jnp.int32)
    eidx3 = eidx.reshape(NE_CHUNKS, KROWS, 128)

    vp, cp = _sc_vertex_sums(vdata, vidx)
    e_full = _tc_edge_full(edata_t, seg_first, seg_last)
    e_mix = _tc_edge_mixed(edata_t, eidx3, ids, valid_arr)
    return _tc_final(cdata, vp, cp, e_full, e_mix, W, b.reshape(1, -1))
